# baseline (speedup 1.0000x reference)
import functools

import numpy as np

import jax
import jax.numpy as jnp
from jax import lax
from jax.experimental import pallas as pl
from jax.experimental.pallas import tpu as pltpu
from jax.experimental.pallas import tpu_sc as plsc

B = 16384
NW = 32
ROWS_W = B // NW
QTR = 128
L = 16
D_OUT = 148
SG = 16
N_SG = QTR // SG

_DENSE = ((12, 0), (3, 12), (13, 15), (12, 72), (3, 84), (13, 87))
_DW = SG * sum(w for w, _ in _DENSE)
_N_DCH = _DW // L
_EMB = ((0, 32, 28), (1, 4, 60), (2, 8, 64),
        (0, 32, 100), (1, 4, 132), (2, 8, 136), (3, 4, 144))


def _dense_patterns():
    rows, cols = [], []
    for w, off in _DENSE:
        s = np.arange(SG * w)
        rows.append(s // w)
        cols.append(off + s % w)
    return (np.concatenate(rows).astype(np.int32),
            np.concatenate(cols).astype(np.int32))

_PDR_HOST, _PDC_HOST = _dense_patterns()

_SET = [
    pltpu.VMEM((N_SG, _DW), jnp.float32),
    pltpu.VMEM((QTR, D_OUT), jnp.float32),
    pltpu.SemaphoreType.DMA,
    pltpu.SemaphoreType.DMA,
]


@functools.partial(
    pl.kernel,
    out_type=jax.ShapeDtypeStruct((B, D_OUT), jnp.float32),
    mesh=plsc.VectorSubcoreMesh(core_axis_name="c", subcore_axis_name="s"),
    compiler_params=pltpu.CompilerParams(
        use_tc_tiling_on_sc=False, needs_layout_passes=False),
    scratch_types=[
        pltpu.VMEM((7 * ROWS_W,), jnp.int32),
        pltpu.VMEM((_DW,), jnp.int32),
        pltpu.VMEM((_DW,), jnp.int32),
        pltpu.VMEM((400, 32), jnp.float32),
        pltpu.VMEM((8, 8), jnp.float32),
        pltpu.VMEM((33, 8), jnp.float32),
        pltpu.VMEM((33, 8), jnp.float32),
        pltpu.SemaphoreType.DMA,
    ] + _SET + _SET,
)
def _encode_sc(p0c, p0b, p0k, p1c, p1b, p1k,
               i_p0a, i_p0j, i_p0c, i_p1a, i_p1j, i_p1c, i_stg,
               t_act, t_jmp, t_chr, t_stg, pdr_hbm, pdc_hbm,
               out_hbm,
               idxv, pdr, pdc, va, vj, vc, vs, psem, *sets):
    setA, setB = sets[:4], sets[4:]
    vtabs = (va, vj, vc, vs)
    wid = lax.axis_index("s") * 2 + lax.axis_index("c")
    base = wid * ROWS_W
    drow = wid * 32

    pcps = [pltpu.async_copy(pdr_hbm, pdr, psem),
            pltpu.async_copy(pdc_hbm, pdc, psem)]
    for src, dst in zip((t_act, t_jmp, t_chr, t_stg), vtabs):
        pcps.append(pltpu.async_copy(src, dst, psem))
    for f, ih in enumerate((i_p0a, i_p0j, i_p0c, i_p1a, i_p1j, i_p1c, i_stg)):
        pcps.append(pltpu.async_copy(
            ih.at[pl.ds(base, ROWS_W)],
            idxv.at[pl.ds(f * ROWS_W, ROWS_W)], psem))
    for cp in pcps:
        cp.wait()

    dsrcs = (p0c, p0b, p0k, p1c, p1b, p1k)
    iota = lax.iota(jnp.int32, L)

    def fire(q, S):
        sd, gsem = S[0], S[2]
        cps = []
        doff = 0
        for dsrc, (w, _) in zip(dsrcs, _DENSE):
            cps.append(pltpu.async_copy(
                dsrc.at[pl.ds(drow + q * N_SG, N_SG), :],
                sd.at[:, pl.ds(doff, SG * w)], gsem))
            doff += SG * w
        return cps

    def assemble(q, S):
        sd, outb = S[0], S[1]

        def dense_sg(g, carry):
            g16 = g * SG

            @plsc.parallel_loop(0, _N_DCH, unroll=8)
            def dense_chunk(u):
                rvec = pdr[pl.ds(u * L, L)] + g16
                cvec = pdc[pl.ds(u * L, L)]
                vals = sd[g, pl.ds(u * L, L)]
                plsc.store_scatter(outb, [rvec, cvec], vals)
            return carry

        lax.fori_loop(0, N_SG, dense_sg, 0)

        for f, (tid, w, off) in enumerate(_EMB):
            tab = vtabs[tid]
            lw = w.bit_length() - 1
            fbase = f * ROWS_W + q * QTR

            @plsc.parallel_loop(0, QTR * w // L, unroll=8)
            def emb_chunk(k, tab=tab, w=w, off=off, lw=lw, fbase=fbase):
                svec = k * L + iota
                rvec = lax.shift_right_logical(svec, lw)
                cvec = lax.bitwise_and(svec, w - 1)
                ivec = plsc.load_gather(idxv, [rvec + fbase])
                vals = plsc.load_gather(tab, [ivec, cvec])
                plsc.store_scatter(outb, [rvec, cvec + off], vals)

    stage_cps = {0: fire(0, setA)}
    wb = {}
    for q in range(4):
        S = (setA, setB)[q % 2]
        if q + 1 < 4:
            stage_cps[q + 1] = fire(q + 1, (setA, setB)[(q + 1) % 2])
        for cp in stage_cps.pop(q):
            cp.wait()
        if q >= 2:
            wb[q - 2].wait()
        assemble(q, S)
        wb[q] = pltpu.async_copy(
            S[1], out_hbm.at[pl.ds(base + q * QTR, QTR), :], S[3])
    wb[2].wait()
    wb[3].wait()


def kernel(p0_continuous, p0_binary, p0_controller, p0_action, p0_jumps,
           p0_character, p1_continuous, p1_binary, p1_controller, p1_action,
           p1_jumps, p1_character, stage, action_table, jumps_table,
           char_table, stage_table):
    def idx(a):
        return a.astype(jnp.int32)

    def dense(a, w):
        return a.reshape(B // SG, SG * w)
    return _encode_sc(
        dense(p0_continuous, 12), dense(p0_binary, 3),
        dense(p0_controller, 13), dense(p1_continuous, 12),
        dense(p1_binary, 3), dense(p1_controller, 13),
        idx(p0_action), idx(p0_jumps), idx(p0_character),
        idx(p1_action), idx(p1_jumps), idx(p1_character), idx(stage),
        action_table, jnp.pad(jumps_table, ((0, 0), (0, 4))), char_table,
        jnp.pad(stage_table, ((0, 0), (0, 4))),
        jnp.asarray(_PDR_HOST), jnp.asarray(_PDC_HOST))
